# baseline (speedup 1.0000x reference)
import functools

import jax
import jax.numpy as jnp
from jax import lax
from jax.experimental import pallas as pl
from jax.experimental.pallas import tpu as pltpu
from jax.experimental.pallas import tpu_sc as plsc

B = 16384
N_CAT = 26
VOCAB = 100
N_CONT = 13
EMB = 19
OUT_W = (N_CAT + 1) * EMB
TBL_W = N_CAT * VOCAB * EMB

NC, NS = 2, 16
NW = NC * NS
EPW = B // NW
E = 32
NCH = EPW // E
CONT_W = EPW * EMB
WIDX_W = EPW * (N_CAT + 1)
CHUNK_W = E * OUT_W

BLK = 512


def _prep_body(cat_ref, num_ref, w_ref, b_ref, cont_ref, len_ref, idx_ref):
    cat = cat_ref[...]
    num = num_ref[...]
    cont_ref[...] = (
        jnp.dot(num, w_ref[...].T, preferred_element_type=jnp.float32)
        + b_ref[...]
    )
    len_ref[...] = jnp.sum((cat != 0).astype(jnp.int32), axis=1, keepdims=True)
    f = lax.broadcasted_iota(jnp.int32, (BLK, N_CAT), 1)
    rows = lax.broadcasted_iota(jnp.int32, (BLK, 1), 0)
    idx_ref[...] = jnp.concatenate(
        [cat * EMB + 1881 * f, TBL_W - 494 + EMB * rows], axis=1
    )


def _tc_prep(cat, num, W, b2d):
    grid = B // BLK
    return pl.pallas_call(
        _prep_body,
        grid=(grid,),
        in_specs=[
            pl.BlockSpec((BLK, N_CAT), lambda i: (i, 0)),
            pl.BlockSpec((BLK, N_CONT), lambda i: (i, 0)),
            pl.BlockSpec((EMB, N_CONT), lambda i: (0, 0)),
            pl.BlockSpec((1, EMB), lambda i: (0, 0)),
        ],
        out_specs=[
            pl.BlockSpec((BLK, EMB), lambda i: (i, 0)),
            pl.BlockSpec((BLK, 1), lambda i: (i, 0)),
            pl.BlockSpec((BLK, N_CAT + 1), lambda i: (i, 0)),
        ],
        out_shape=[
            jax.ShapeDtypeStruct((B, EMB), jnp.float32),
            jax.ShapeDtypeStruct((B, 1), jnp.int32),
            jax.ShapeDtypeStruct((B, N_CAT + 1), jnp.int32),
        ],
    )(cat, num, W, b2d)


_sc_mesh = plsc.VectorSubcoreMesh(
    core_axis_name="c", subcore_axis_name="s", num_cores=NC, num_subcores=NS
)


@functools.partial(
    pl.kernel,
    out_type=jax.ShapeDtypeStruct((B * OUT_W,), jnp.float32),
    mesh=_sc_mesh,
    compiler_params=pltpu.CompilerParams(
        use_tc_tiling_on_sc=False, needs_layout_passes=False
    ),
    scratch_types=[
        pltpu.VMEM((59640,), jnp.float32),
        pltpu.VMEM((13832,), jnp.int32),
        pltpu.VMEM((CHUNK_W,), jnp.float32),
        pltpu.VMEM((CHUNK_W,), jnp.float32),
        pltpu.SemaphoreType.DMA,
        pltpu.SemaphoreType.DMA,
    ],
)
def _sc_gather(tbl_hbm, cont_hbm, widx_hbm, out_hbm,
               tblbuf, widx_v, obuf_a, obuf_b, sem_a, sem_b):
    wid = lax.axis_index("s") * NC + lax.axis_index("c")
    pltpu.sync_copy(tbl_hbm, tblbuf.at[pl.ds(0, TBL_W)])
    pltpu.sync_copy(cont_hbm.at[pl.ds(wid * CONT_W, CONT_W)],
                    tblbuf.at[pl.ds(TBL_W, CONT_W)])
    pltpu.sync_copy(widx_hbm.at[pl.ds(wid * WIDX_W, WIDX_W)],
                    widx_v.at[pl.ds(0, WIDX_W)])

    iota = lax.iota(jnp.int32, 16)
    mask0 = iota < 1
    obufs = (obuf_a, obuf_b)
    sems = (sem_a, sem_b)
    out_base = wid * EPW * OUT_W

    @pl.loop(0, NCH // 2)
    def _pair(ch2):
        for par in range(2):
            ch = ch2 * 2 + par
            obuf, sem = obufs[par], sems[par]
            dst = out_hbm.at[pl.ds(out_base + ch * CHUNK_W, CHUNK_W)]

            @pl.when(ch2 > 0)
            def _wait_prev():
                pltpu.make_async_copy(obuf, dst, sem).wait()

            @plsc.parallel_loop(0, E, unroll=16)
            def _ex(b):
                eb = (ch * E + b) * (N_CAT + 1)
                ob = b * OUT_W
                for c in range(33):
                    pv = iota + (16 * c)
                    fvec = (pv * 55189) >> 20
                    if c < 32:
                        bases = plsc.load_gather(widx_v, [eb + fvec])
                        vals = plsc.load_gather(tblbuf, [bases + pv])
                        plsc.store_scatter(obuf, [ob + pv], vals)
                    else:
                        bases = plsc.load_gather(widx_v, [eb + fvec], mask=mask0)
                        vals = plsc.load_gather(tblbuf, [bases + pv], mask=mask0)
                        plsc.store_scatter(obuf, [ob + pv], vals, mask=mask0)

            pltpu.make_async_copy(obuf, dst, sem).start()

    for par in range(2):
        last_ch = NCH - 2 + par
        dst = out_hbm.at[pl.ds(out_base + last_ch * CHUNK_W, CHUNK_W)]
        pltpu.make_async_copy(obufs[par], dst, sems[par]).wait()


def kernel(cat, num, constraints, tables, W, b):
    del constraints
    cont, lengths2d, widx = _tc_prep(cat, num, W, b.reshape(1, EMB))
    flat = _sc_gather(
        tables.reshape(TBL_W),
        cont.reshape(B * EMB),
        widx.reshape(B * (N_CAT + 1)),
    )
    return flat.reshape(B, OUT_W), lengths2d.reshape(B)


# speedup vs baseline: 1.2931x; 1.2931x over previous
import functools

import jax
import jax.numpy as jnp
from jax import lax
from jax.experimental import pallas as pl
from jax.experimental.pallas import tpu as pltpu
from jax.experimental.pallas import tpu_sc as plsc

B = 16384
N_CAT = 26
VOCAB = 100
N_CONT = 13
EMB = 19
OUT_W = (N_CAT + 1) * EMB
TBL_W = N_CAT * VOCAB * EMB

NC, NS = 2, 16
NW = NC * NS
EPW = B // NW
E = 32
NCH = EPW // E
CONT_W = EPW * EMB
WIDX_W = EPW * (N_CAT + 1)
CHUNK_W = E * OUT_W

BLK = 512


def _prep_body(cat_ref, num_ref, w_ref, b_ref, cont_ref, len_ref, idx_ref):
    cat = cat_ref[...]
    num = num_ref[...]
    cont_ref[...] = (
        jnp.dot(num, w_ref[...].T, preferred_element_type=jnp.float32)
        + b_ref[...]
    )
    len_ref[...] = jnp.sum((cat != 0).astype(jnp.int32), axis=1, keepdims=True)
    f = lax.broadcasted_iota(jnp.int32, (BLK, N_CAT), 1)
    rows = lax.broadcasted_iota(jnp.int32, (BLK, 1), 0)
    idx_ref[...] = jnp.concatenate(
        [cat * EMB + 1881 * f, TBL_W - 494 + EMB * rows], axis=1
    )


def _tc_prep(cat, num, W, b2d):
    grid = B // BLK
    return pl.pallas_call(
        _prep_body,
        grid=(grid,),
        in_specs=[
            pl.BlockSpec((BLK, N_CAT), lambda i: (i, 0)),
            pl.BlockSpec((BLK, N_CONT), lambda i: (i, 0)),
            pl.BlockSpec((EMB, N_CONT), lambda i: (0, 0)),
            pl.BlockSpec((1, EMB), lambda i: (0, 0)),
        ],
        out_specs=[
            pl.BlockSpec((BLK, EMB), lambda i: (i, 0)),
            pl.BlockSpec((BLK, 1), lambda i: (i, 0)),
            pl.BlockSpec((BLK, N_CAT + 1), lambda i: (i, 0)),
        ],
        out_shape=[
            jax.ShapeDtypeStruct((B, EMB), jnp.float32),
            jax.ShapeDtypeStruct((B, 1), jnp.int32),
            jax.ShapeDtypeStruct((B, N_CAT + 1), jnp.int32),
        ],
    )(cat, num, W, b2d)


_sc_mesh = plsc.VectorSubcoreMesh(
    core_axis_name="c", subcore_axis_name="s", num_cores=NC, num_subcores=NS
)


@functools.partial(
    pl.kernel,
    out_type=jax.ShapeDtypeStruct((B * OUT_W,), jnp.float32),
    mesh=_sc_mesh,
    compiler_params=pltpu.CompilerParams(
        use_tc_tiling_on_sc=False, needs_layout_passes=False
    ),
    scratch_types=[
        pltpu.VMEM((59640,), jnp.float32),
        pltpu.VMEM((13832,), jnp.int32),
        pltpu.VMEM((CHUNK_W,), jnp.float32),
        pltpu.VMEM((CHUNK_W,), jnp.float32),
        pltpu.SemaphoreType.DMA,
        pltpu.SemaphoreType.DMA,
    ],
)
def _sc_gather(tbl_hbm, cont_hbm, widx_hbm, out_hbm,
               tblbuf, widx_v, obuf_a, obuf_b, sem_a, sem_b):
    wid = lax.axis_index("s") * NC + lax.axis_index("c")
    pltpu.sync_copy(tbl_hbm, tblbuf.at[pl.ds(0, TBL_W)])
    pltpu.sync_copy(cont_hbm.at[pl.ds(wid * CONT_W, CONT_W)],
                    tblbuf.at[pl.ds(TBL_W, CONT_W)])
    pltpu.sync_copy(widx_hbm.at[pl.ds(wid * WIDX_W, WIDX_W)],
                    widx_v.at[pl.ds(0, WIDX_W)])

    iota = lax.iota(jnp.int32, 16)
    mask0 = iota < 1
    obufs = (obuf_a, obuf_b)
    sems = (sem_a, sem_b)
    out_base = wid * EPW * OUT_W

    @pl.loop(0, NCH // 2)
    def _pair(ch2):
        for par in range(2):
            ch = ch2 * 2 + par
            obuf, sem = obufs[par], sems[par]
            dst = out_hbm.at[pl.ds(out_base + ch * CHUNK_W, CHUNK_W)]

            @pl.when(ch2 > 0)
            def _wait_prev():
                pltpu.make_async_copy(obuf, dst, sem).wait()

            @plsc.parallel_loop(0, E, unroll=8)
            def _ex(b):
                eb = (ch * E + b) * (N_CAT + 1)
                ob = b * OUT_W
                for c in range(33):
                    pv = iota + (16 * c)
                    fvec = (pv * 55189) >> 20
                    if c < 32:
                        bases = plsc.load_gather(widx_v, [eb + fvec])
                        vals = plsc.load_gather(tblbuf, [bases + pv])
                        obuf[pl.ds(ob + 16 * c, 16)] = vals
                    else:
                        bases = plsc.load_gather(widx_v, [eb + fvec], mask=mask0)
                        vals = plsc.load_gather(tblbuf, [bases + pv], mask=mask0)
                        plsc.store_scatter(obuf, [ob + pv], vals, mask=mask0)

            pltpu.make_async_copy(obuf, dst, sem).start()

    for par in range(2):
        last_ch = NCH - 2 + par
        dst = out_hbm.at[pl.ds(out_base + last_ch * CHUNK_W, CHUNK_W)]
        pltpu.make_async_copy(obufs[par], dst, sems[par]).wait()


def kernel(cat, num, constraints, tables, W, b):
    del constraints
    cont, lengths2d, widx = _tc_prep(cat, num, W, b.reshape(1, EMB))
    flat = _sc_gather(
        tables.reshape(TBL_W),
        cont.reshape(B * EMB),
        widx.reshape(B * (N_CAT + 1)),
    )
    return flat.reshape(B, OUT_W), lengths2d.reshape(B)


# speedup vs baseline: 1.5738x; 1.2171x over previous
import functools

import jax
import jax.numpy as jnp
from jax import lax
from jax.experimental import pallas as pl
from jax.experimental.pallas import tpu as pltpu
from jax.experimental.pallas import tpu_sc as plsc

B = 16384
N_CAT = 26
VOCAB = 100
N_CONT = 13
EMB = 19
OUT_W = (N_CAT + 1) * EMB
TBL_W = N_CAT * VOCAB * EMB
TBL_PAD = 49408
CONT_COL = 32

NC, NS = 2, 16
NW = NC * NS
EPW = B // NW
E = 32
NCH = EPW // E
CHUNK_W = E * OUT_W

BLK = 2048


def _prep_body(cat_ref, num_ref, wt_ref, b_ref, x_ref, len_ref):
    cat = cat_ref[...]
    num = num_ref[...]
    cont = (
        jnp.dot(num, wt_ref[...], preferred_element_type=jnp.float32)
        + b_ref[...]
    )
    f = lax.broadcasted_iota(jnp.int32, (BLK, N_CAT), 1)
    x_ref[...] = jnp.concatenate(
        [
            cat * EMB + 1881 * f,
            jnp.zeros((BLK, CONT_COL - N_CAT), jnp.int32),
            lax.bitcast_convert_type(cont, jnp.int32),
            jnp.zeros((BLK, 128 - CONT_COL - EMB), jnp.int32),
        ],
        axis=1,
    )
    len_ref[...] = jnp.sum((cat != 0).astype(jnp.int32), axis=1, keepdims=True)


def _tc_prep(cat, num, Wt, b2d):
    grid = B // BLK
    return pl.pallas_call(
        _prep_body,
        grid=(grid,),
        in_specs=[
            pl.BlockSpec((BLK, N_CAT), lambda i: (i, 0)),
            pl.BlockSpec((BLK, N_CONT), lambda i: (i, 0)),
            pl.BlockSpec((N_CONT, EMB), lambda i: (0, 0)),
            pl.BlockSpec((1, EMB), lambda i: (0, 0)),
        ],
        out_specs=[
            pl.BlockSpec((BLK, 128), lambda i: (i, 0)),
            pl.BlockSpec((BLK, 1), lambda i: (i, 0)),
        ],
        out_shape=[
            jax.ShapeDtypeStruct((B, 128), jnp.int32),
            jax.ShapeDtypeStruct((B, 1), jnp.int32),
        ],
    )(cat, num, Wt, b2d)


_sc_mesh = plsc.VectorSubcoreMesh(
    core_axis_name="c", subcore_axis_name="s", num_cores=NC, num_subcores=NS
)


@functools.partial(
    pl.kernel,
    out_type=jax.ShapeDtypeStruct((B * OUT_W,), jnp.float32),
    mesh=_sc_mesh,
    compiler_params=pltpu.CompilerParams(
        use_tc_tiling_on_sc=False, needs_layout_passes=False
    ),
    scratch_types=[
        pltpu.VMEM((TBL_PAD,), jnp.float32),
        pltpu.VMEM((E, 128), jnp.int32),
        pltpu.VMEM((E, 128), jnp.int32),
        pltpu.VMEM((CHUNK_W,), jnp.float32),
        pltpu.VMEM((CHUNK_W,), jnp.float32),
        pltpu.SemaphoreType.DMA,
        pltpu.SemaphoreType.DMA,
        pltpu.SemaphoreType.DMA,
        pltpu.SemaphoreType.DMA,
    ],
)
def _sc_gather(tbl_hbm, x_hbm, out_hbm,
               tblbuf, xb_a, xb_b, obuf_a, obuf_b,
               osem_a, osem_b, xsem_a, xsem_b):
    wid = lax.axis_index("s") * NC + lax.axis_index("c")
    ex0 = wid * EPW

    pltpu.sync_copy(tbl_hbm, tblbuf)

    iota = lax.iota(jnp.int32, 16)
    mask30 = iota < 14
    xbufs = (xb_a, xb_b)
    xsems = (xsem_a, xsem_b)
    obufs = (obuf_a, obuf_b)
    osems = (osem_a, osem_b)
    out_base = wid * EPW * OUT_W

    pltpu.async_copy(x_hbm.at[pl.ds(ex0, E)], xb_a, xsem_a)

    @pl.loop(0, NCH // 2)
    def _pair(ch2):
        for par in range(2):
            ch = ch2 * 2 + par
            obuf, osem = obufs[par], osems[par]
            xbuf = xbufs[par]
            dst = out_hbm.at[pl.ds(out_base + ch * CHUNK_W, CHUNK_W)]

            @pl.when(ch < NCH - 1)
            def _prefetch():
                pltpu.async_copy(
                    x_hbm.at[pl.ds(ex0 + (ch + 1) * E, E)],
                    xbufs[1 - par], xsems[1 - par],
                )

            pltpu.make_async_copy(
                x_hbm.at[pl.ds(ex0 + ch * E, E)], xbuf, xsems[par]
            ).wait()

            @pl.when(ch2 > 0)
            def _wait_prev():
                pltpu.make_async_copy(obuf, dst, osem).wait()

            @plsc.parallel_loop(0, E, unroll=8)
            def _ex(b):
                ob = b * OUT_W
                bvec = jnp.full((16,), 0, jnp.int32) + b
                for c in range(31):
                    pv = iota + (16 * c)
                    fvec = (pv * 55189) >> 20
                    bases = plsc.load_gather(xbuf, [bvec, fvec])
                    vals = plsc.load_gather(tblbuf, [bases + pv])
                    if c < 30:
                        obuf[pl.ds(ob + 16 * c, 16)] = vals
                    else:
                        plsc.store_scatter(obuf, [ob + pv], vals, mask=mask30)
                c0 = xbuf[b, pl.ds(CONT_COL, 16)]
                c1 = xbuf[b, pl.ds(CONT_COL + 3, 16)]
                obuf[pl.ds(ob + 494, 16)] = plsc.bitcast(c0, jnp.float32)
                obuf[pl.ds(ob + 497, 16)] = plsc.bitcast(c1, jnp.float32)

            pltpu.make_async_copy(obuf, dst, osem).start()

    for par in range(2):
        last_ch = NCH - 2 + par
        dst = out_hbm.at[pl.ds(out_base + last_ch * CHUNK_W, CHUNK_W)]
        pltpu.make_async_copy(obufs[par], dst, osems[par]).wait()


def kernel(cat, num, constraints, tables, W, b):
    del constraints
    x, lengths2d = _tc_prep(cat, num, W.T, b.reshape(1, EMB))
    tbl1 = jnp.concatenate(
        [tables.reshape(TBL_W), jnp.zeros((TBL_PAD - TBL_W,), jnp.float32)]
    )
    flat = _sc_gather(tbl1, x)
    return flat.reshape(B, OUT_W), lengths2d.reshape(B)
